# Initial kernel scaffold; baseline (speedup 1.0000x reference)
import jax
import jax.numpy as jnp
from jax.experimental import pallas as pl


def kernel(index, targets, table):
    raise NotImplementedError("write your pallas kernel here")


# speedup vs baseline: 1.3707x; 1.3707x over previous
import functools

import jax
import jax.numpy as jnp
from jax import lax
from jax.experimental import pallas as pl
from jax.experimental.pallas import tpu as pltpu
from jax.experimental.pallas import tpu_sc as plsc

V = 1000
LSE_PAD = 1024
NUM_CORES = 2
NUM_SUBCORES = 16
LANES = 16
NW = NUM_CORES * NUM_SUBCORES


def _lse_body(table_ref, out_ref):
    x = table_ref[...]
    m = jnp.max(x, axis=1, keepdims=True)
    s = jnp.sum(jnp.exp(x - m), axis=1, keepdims=True)
    lse = m + jnp.log(s)
    pad = jnp.zeros((LSE_PAD - V, 1), jnp.float32)
    out_ref[...] = jnp.concatenate([lse, pad], axis=0)


def _compute_lse(table):
    out = pl.pallas_call(
        _lse_body,
        out_shape=jax.ShapeDtypeStruct((LSE_PAD, 1), jnp.float32),
    )(table)
    return out.reshape(LSE_PAD)


def _loss_body(part_ref, out_ref, *, n_tokens):
    val = jnp.sum(part_ref[...]) * (1.0 / n_tokens)
    out_ref[...] = jnp.broadcast_to(val, (1, 1))


def _compute_loss(partials, n_tokens):
    out = pl.pallas_call(
        functools.partial(_loss_body, n_tokens=n_tokens),
        out_shape=jax.ShapeDtypeStruct((1, 1), jnp.float32),
    )(partials)
    return out[0, 0]


def _make_sc_gather(n_tokens):
    assert n_tokens % (8 * NW) == 0
    per_w = n_tokens // NW
    chunk = 64
    assert per_w % chunk == 0
    n_chunks = per_w // chunk

    mesh = plsc.VectorSubcoreMesh(
        core_axis_name="c", subcore_axis_name="s",
        num_cores=NUM_CORES, num_subcores=NUM_SUBCORES)

    @functools.partial(
        pl.kernel,
        mesh=mesh,
        compiler_params=pltpu.CompilerParams(
            use_tc_tiling_on_sc=False, needs_layout_passes=False),
        out_type=[
            jax.ShapeDtypeStruct((n_tokens, V), jnp.float32),
            jax.ShapeDtypeStruct((NW, LANES), jnp.float32),
        ],
        scratch_types=[
            pltpu.VMEM((chunk,), jnp.int32),
            pltpu.VMEM((chunk,), jnp.int32),
            pltpu.VMEM((chunk, V), jnp.float32),
            pltpu.VMEM((LANES,), jnp.float32),
            pltpu.VMEM((LSE_PAD,), jnp.float32),
            pltpu.SemaphoreType.DMA,
        ],
    )
    def sc_gather(table_hbm, idx_hbm, tgt_hbm, lse_hbm, out_hbm, part_hbm,
                  idx_v, tgt_v, rows_v, acc_v, lse_v, sem):
        wid = lax.axis_index("s") * NUM_CORES + lax.axis_index("c")
        pltpu.sync_copy(lse_hbm, lse_v)
        acc_v[...] = jnp.zeros((LANES,), jnp.float32)

        def body(g, carry):
            base = pl.multiple_of(wid * per_w + g * chunk, chunk)
            pltpu.sync_copy(idx_hbm.at[pl.ds(base, chunk)], idx_v)
            pltpu.sync_copy(tgt_hbm.at[pl.ds(base, chunk)], tgt_v)
            pltpu.async_copy(table_hbm.at[idx_v], rows_v, sem).wait()
            pltpu.sync_copy(rows_v, out_hbm.at[pl.ds(base, chunk)])
            for j in range(chunk // LANES):
                idx16 = idx_v[pl.ds(j * LANES, LANES)]
                tgt16 = tgt_v[pl.ds(j * LANES, LANES)]
                lse16 = plsc.load_gather(lse_v, [idx16])
                rid16 = lax.iota(jnp.int32, LANES) + (j * LANES)
                picked = plsc.load_gather(rows_v, [rid16, tgt16])
                acc_v[...] = acc_v[...] + (lse16 - picked)
            return carry

        lax.fori_loop(0, n_chunks, body, 0)
        pltpu.sync_copy(acc_v, part_hbm.at[wid])

    return sc_gather


def kernel(index, targets, table):
    b, t = index.shape
    n_tokens = b * t
    idx_flat = index.reshape(n_tokens).astype(jnp.int32)
    tgt_flat = targets.reshape(n_tokens).astype(jnp.int32)
    lse = _compute_lse(table)
    sc_gather = _make_sc_gather(n_tokens)
    logits_flat, partials = sc_gather(table, idx_flat, tgt_flat, lse)
    loss = _compute_loss(partials, n_tokens)
    return logits_flat.reshape(b, t, V), loss


# speedup vs baseline: 1.4278x; 1.0417x over previous
import functools

import jax
import jax.numpy as jnp
from jax import lax
from jax.experimental import pallas as pl
from jax.experimental.pallas import tpu as pltpu
from jax.experimental.pallas import tpu_sc as plsc

V = 1000
LSE_PAD = 1024
NUM_CORES = 2
NUM_SUBCORES = 16
LANES = 16
NW = NUM_CORES * NUM_SUBCORES


def _lse_body(table_ref, out_ref):
    x = table_ref[...]
    m = jnp.max(x, axis=1, keepdims=True)
    s = jnp.sum(jnp.exp(x - m), axis=1, keepdims=True)
    lse = m + jnp.log(s)
    pad = jnp.zeros((LSE_PAD - V, 1), jnp.float32)
    out_ref[...] = jnp.concatenate([lse, pad], axis=0)


def _compute_lse(table):
    out = pl.pallas_call(
        _lse_body,
        out_shape=jax.ShapeDtypeStruct((LSE_PAD, 1), jnp.float32),
    )(table)
    return out.reshape(LSE_PAD)


def _loss_body(part_ref, out_ref, *, n_tokens):
    val = jnp.sum(part_ref[...]) * (1.0 / n_tokens)
    out_ref[...] = jnp.broadcast_to(val, (1, 1))


def _compute_loss(partials, n_tokens):
    out = pl.pallas_call(
        functools.partial(_loss_body, n_tokens=n_tokens),
        out_shape=jax.ShapeDtypeStruct((1, 1), jnp.float32),
    )(partials)
    return out[0, 0]


def _make_sc_gather(n_tokens):
    assert n_tokens % (8 * NW) == 0
    per_w = n_tokens // NW
    chunk = 32
    nbuf = 2
    assert per_w % chunk == 0
    n_chunks = per_w // chunk
    assert n_chunks % nbuf == 0 and chunk % LANES == 0

    mesh = plsc.VectorSubcoreMesh(
        core_axis_name="c", subcore_axis_name="s",
        num_cores=NUM_CORES, num_subcores=NUM_SUBCORES)

    @functools.partial(
        pl.kernel,
        mesh=mesh,
        compiler_params=pltpu.CompilerParams(
            use_tc_tiling_on_sc=False, needs_layout_passes=False),
        out_type=[
            jax.ShapeDtypeStruct((n_tokens, V), jnp.float32),
            jax.ShapeDtypeStruct((NW, LANES), jnp.float32),
        ],
        scratch_types=(
            [pltpu.VMEM((per_w,), jnp.int32)] * 2
            + [pltpu.VMEM((chunk, V), jnp.float32)] * nbuf
            + [pltpu.VMEM((LANES,), jnp.float32)]
            + [pltpu.VMEM((LSE_PAD,), jnp.float32)]
            + [pltpu.SemaphoreType.DMA] * nbuf
            + [pltpu.SemaphoreType.DMA] * nbuf
        ),
    )
    def sc_gather(table_hbm, idx_hbm, tgt_hbm, lse_hbm, out_hbm, part_hbm,
                  idx_v, tgt_v, *rest):
        rows = rest[:nbuf]
        acc_v, lse_v = rest[nbuf], rest[nbuf + 1]
        gsem = rest[nbuf + 2:nbuf + 2 + nbuf]
        ssem = rest[nbuf + 2 + nbuf:]
        wid = lax.axis_index("s") * NUM_CORES + lax.axis_index("c")
        base_w = pl.multiple_of(wid * per_w, per_w)
        pltpu.sync_copy(lse_hbm, lse_v)
        pltpu.sync_copy(idx_hbm.at[pl.ds(base_w, per_w)], idx_v)
        pltpu.sync_copy(tgt_hbm.at[pl.ds(base_w, per_w)], tgt_v)
        acc_v[...] = jnp.zeros((LANES,), jnp.float32)

        def start_gather(g, b):
            off = pl.multiple_of(g * chunk, chunk)
            pltpu.async_copy(
                table_hbm.at[idx_v.at[pl.ds(off, chunk)]], rows[b], gsem[b])

        def wait_gather(b):
            pltpu.make_async_copy(
                table_hbm.at[pl.ds(0, chunk)], rows[b], gsem[b]).wait()

        def start_scatter(g, b):
            off = pl.multiple_of(base_w + g * chunk, chunk)
            pltpu.async_copy(rows[b], out_hbm.at[pl.ds(off, chunk)], ssem[b])

        def wait_scatter(b):
            pltpu.make_async_copy(
                rows[b], out_hbm.at[pl.ds(0, chunk)], ssem[b]).wait()

        def loss_partial(g, b):
            part = jnp.zeros((LANES,), jnp.float32)
            for j in range(chunk // LANES):
                off = pl.multiple_of(g * chunk + j * LANES, LANES)
                idx16 = idx_v[pl.ds(off, LANES)]
                tgt16 = tgt_v[pl.ds(off, LANES)]
                lse16 = plsc.load_gather(lse_v, [idx16])
                rid16 = lax.iota(jnp.int32, LANES) + (j * LANES)
                part = part + lse16 - plsc.load_gather(rows[b], [rid16, tgt16])
            acc_v[...] = acc_v[...] + part

        for b in range(nbuf):
            start_gather(b, b)

        def body(k, carry):
            for b in range(nbuf):
                g = k * nbuf + b
                wait_gather(b)
                start_scatter(g, b)
                loss_partial(g, b)
                @pl.when(g + nbuf < n_chunks)
                def _():
                    wait_scatter(b)
                    start_gather(g + nbuf, b)
            return carry

        lax.fori_loop(0, n_chunks // nbuf, body, 0)
        for b in range(nbuf):
            wait_scatter(b)
        pltpu.sync_copy(acc_v, part_hbm.at[wid])

    return sc_gather


def kernel(index, targets, table):
    b, t = index.shape
    n_tokens = b * t
    idx_flat = index.reshape(n_tokens).astype(jnp.int32)
    tgt_flat = targets.reshape(n_tokens).astype(jnp.int32)
    lse = _compute_lse(table)
    sc_gather = _make_sc_gather(n_tokens)
    logits_flat, partials = sc_gather(table, idx_flat, tgt_flat, lse)
    loss = _compute_loss(partials, n_tokens)
    return logits_flat.reshape(b, t, V), loss
